# Initial kernel scaffold; baseline (speedup 1.0000x reference)
import jax
import jax.numpy as jnp
from jax.experimental import pallas as pl


def kernel(inputs, edge_index, edge_values, initial_hidden_state, w_gate_0, b_gate_0, w_cand_0, b_cand_0, w_gate_1, b_gate_1, w_cand_1, b_cand_1):
    raise NotImplementedError("write your pallas kernel here")


# speedup vs baseline: 6.8517x; 6.8517x over previous
import functools

import jax
import jax.numpy as jnp
from jax import lax
from jax.experimental import pallas as pl
from jax.experimental.pallas import tpu as pltpu
from jax.experimental.pallas import tpu_sc as plsc

SEQ = 6
B = 2
N = 50000
E = 800000
IN_DIM = 2
HID = 16
KDIFF = 2
NM = KDIFF + 1
WP = 32

NS = 16
NC = 2
RPT_OUT = 3128
CW = 400
RB = 8
ER = E // CW
NBLOCK = ER // RB
MAXBLK = 16
ZR = 136


def _spmm_body(x_ref, src_ref, dst_ref, w_ref, out_ref,
               acc, srcbuf, dstbuf, wbuf, rows, sem):
    cid = lax.axis_index("c")
    sid = lax.axis_index("s")

    zeros16 = jnp.zeros((16,), jnp.float32)
    for i in range(ZR):
        for q in range(WP // 16):
            rows[i, pl.ds(q * 16, 16)] = zeros16

    zsrc = rows.at[pl.ds(0, ZR)]
    z0 = pl.multiple_of(sid * RPT_OUT, 8)
    for k in range(22):
        pltpu.sync_copy(zsrc, acc.at[pl.ds(z0 + k * ZR, ZR)])

    @pl.when(sid < NS - 1)
    def _():
        pltpu.sync_copy(zsrc, acc.at[pl.ds(z0 + 22 * ZR, ZR)])

    @pl.when(sid == NS - 1)
    def _():
        pltpu.sync_copy(rows.at[pl.ds(0, 88)],
                        acc.at[pl.ds((NS - 1) * RPT_OUT + 22 * ZR, 88)])

    plsc.subcore_barrier()

    x_my = x_ref.at[cid]
    out_my = out_ref.at[cid]

    def edge_block(k, _):
        base = pl.multiple_of((sid + k * NS) * RB, 8)
        pltpu.sync_copy(src_ref.at[pl.ds(base, RB)], srcbuf)
        pltpu.sync_copy(dst_ref.at[pl.ds(base, RB)], dstbuf)
        pltpu.sync_copy(w_ref.at[pl.ds(base, RB)], wbuf)
        for j in range(RB):
            pltpu.async_copy(x_my.at[srcbuf.at[j]], rows, sem).wait()

            @plsc.parallel_loop(0, CW, 16)
            def _scale(i):
                wv16 = wbuf[j, pl.ds(i, 16)]
                for l in range(16):
                    wv = wv16[l]
                    for q in range(WP // 16):
                        rows[i + l, pl.ds(q * 16, 16)] = (
                            rows[i + l, pl.ds(q * 16, 16)] * wv)

            pltpu.sync_copy(rows, acc.at[dstbuf.at[j]], add=True)
        return 0

    nblk = jnp.where(sid < NBLOCK - NS * (MAXBLK - 1), MAXBLK, MAXBLK - 1)
    lax.fori_loop(0, nblk, edge_block, 0)

    plsc.subcore_barrier()

    @pl.when(sid < NS - 1)
    def _():
        r0 = pl.multiple_of(sid * RPT_OUT, 8)
        pltpu.sync_copy(acc.at[pl.ds(r0, RPT_OUT)],
                        out_my.at[pl.ds(r0, RPT_OUT)])

    @pl.when(sid == NS - 1)
    def _():
        r0 = (NS - 1) * RPT_OUT
        pltpu.sync_copy(acc.at[pl.ds(r0, N - r0)],
                        out_my.at[pl.ds(r0, N - r0)])


@jax.jit
def _spmm(x, src2, dst2, w2):
    mesh = plsc.VectorSubcoreMesh(core_axis_name="c", subcore_axis_name="s",
                                  num_cores=NC, num_subcores=NS)
    f = pl.kernel(
        _spmm_body,
        out_type=jax.ShapeDtypeStruct((B, N, WP), jnp.float32),
        mesh=mesh,
        scratch_types=[
            pltpu.VMEM_SHARED((N, WP), jnp.float32),
            pltpu.VMEM((RB, CW), jnp.int32),
            pltpu.VMEM((RB, CW), jnp.int32),
            pltpu.VMEM((RB, CW), jnp.float32),
            pltpu.VMEM((CW, WP), jnp.float32),
            pltpu.SemaphoreType.DMA,
        ],
        compiler_params=pltpu.CompilerParams(use_tc_tiling_on_sc=False),
    )
    return f(x, src2, dst2, w2)


def _repack_weight(W, d_in):
    I = d_in + HID
    out_dim = W.shape[1]
    Wp = jnp.zeros((NM * WP, out_dim), jnp.float32)
    Wr = W.reshape(I, NM, out_dim)
    for m in range(NM):
        Wp = Wp.at[m * WP: m * WP + I].set(Wr[:, m])
    return Wp


def _dconv(x0, src2, dst2, w2, Wp, bias):
    x1 = _spmm(x0, src2, dst2, w2)
    x2 = 2.0 * _spmm(x1, src2, dst2, w2) - x0
    feats = jnp.concatenate([x0, x1, x2], axis=-1)
    return feats @ Wp + bias


def _pad_concat(inp, state):
    d = inp.shape[2]
    pad = WP - d - HID
    return jnp.concatenate(
        [inp, state, jnp.zeros((B, N, pad), jnp.float32)], axis=2)


def _cell(x, h, src2, dst2, w2, wgp, bg, wcp, bc):
    x0g = _pad_concat(x, h)
    val = jax.nn.sigmoid(_dconv(x0g, src2, dst2, w2, wgp, bg))
    r = val[..., :HID]
    u = val[..., HID:]
    x0c = _pad_concat(x, r * h)
    c = jnp.tanh(_dconv(x0c, src2, dst2, w2, wcp, bc))
    return u * h + (1.0 - u) * c


def kernel(inputs, edge_index, edge_values, initial_hidden_state,
           w_gate_0, b_gate_0, w_cand_0, b_cand_0,
           w_gate_1, b_gate_1, w_cand_1, b_cand_1):
    src2 = edge_index[1].reshape(ER, CW)
    dst2 = edge_index[0].reshape(ER, CW)
    w2 = edge_values.reshape(ER, CW)

    layers = [
        (IN_DIM, _repack_weight(w_gate_0, IN_DIM), b_gate_0,
         _repack_weight(w_cand_0, IN_DIM), b_cand_0),
        (HID, _repack_weight(w_gate_1, HID), b_gate_1,
         _repack_weight(w_cand_1, HID), b_cand_1),
    ]

    current = inputs.reshape(SEQ, B, N, IN_DIM)
    output_hidden = []
    for li, (d_in, wgp, bg, wcp, bc) in enumerate(layers):
        h = initial_hidden_state[li].reshape(B, N, HID)
        outs = []
        for t in range(SEQ):
            h = _cell(current[t], h, src2, dst2, w2, wgp, bg, wcp, bc)
            outs.append(h)
        output_hidden.append(h.reshape(B, N * HID))
        current = jnp.stack(outs, axis=0)
    final = jnp.stack(output_hidden, axis=0)
    seq_out = current.reshape(SEQ, B, N * HID)
    return final, seq_out
